# Initial kernel scaffold; baseline (speedup 1.0000x reference)
import jax
import jax.numpy as jnp
from jax.experimental import pallas as pl


def kernel(x, w_gate, fc1_w, fc1_b, fc2_w, fc2_b):
    raise NotImplementedError("write your pallas kernel here")


# speedup vs baseline: 2.0553x; 2.0553x over previous
import functools

import jax
import jax.numpy as jnp
import numpy as np
from jax.experimental import pallas as pl
from jax.experimental.pallas import tpu as pltpu

B, D, H, O, E, K = 2048, 1024, 2048, 1024, 8, 2
EPS = float(np.finfo(np.float64).eps)

TM = 256


def _router_body(x_ref, wg_ref, gates_ref):
    x = x_ref[...]
    wg = wg_ref[...]
    logits = jnp.dot(x, wg, preferred_element_type=jnp.float32)
    eidx = jax.lax.broadcasted_iota(jnp.int32, logits.shape, 1)
    v1 = jnp.max(logits, axis=1, keepdims=True)
    i1 = jnp.min(jnp.where(logits == v1, eidx, E), axis=1, keepdims=True)
    masked = jnp.where(eidx == i1, -jnp.inf, logits)
    v2 = jnp.max(masked, axis=1, keepdims=True)
    i2 = jnp.min(jnp.where(masked == v2, eidx, E), axis=1, keepdims=True)
    g1 = 1.0 / (1.0 + jnp.exp(v2 - v1))
    g2 = 1.0 / (1.0 + jnp.exp(v1 - v2))
    gates = jnp.where(eidx == i1, g1, 0.0) + jnp.where(eidx == i2, g2, 0.0)
    gates_ref[...] = gates


def _moe_body(x_ref, fc1w_ref, fc1b_ref, fc2w_ref, fc2b_ref, gates_ref, out_ref):
    e = pl.program_id(0)
    x = x_ref[...]
    h = jnp.dot(x, fc1w_ref[0], preferred_element_type=jnp.float32)
    h = h + fc1b_ref[0]
    h = 0.5 * h * (1.0 + jax.lax.erf(h * np.float32(1.0 / np.sqrt(2.0))))
    out = jnp.dot(h.astype(fc2w_ref.dtype), fc2w_ref[0],
                  preferred_element_type=jnp.float32)
    out = out + fc2b_ref[0]
    m = jnp.max(out, axis=1, keepdims=True)
    p = jnp.exp(out - m)
    soft = p / jnp.sum(p, axis=1, keepdims=True)
    gates = gates_ref[...]
    eidx = jax.lax.broadcasted_iota(jnp.int32, gates.shape, 1)
    g = jnp.sum(jnp.where(eidx == e, gates, 0.0), axis=1, keepdims=True)
    contrib = soft * g

    @pl.when(e == 0)
    def _():
        out_ref[...] = contrib

    @pl.when(jnp.logical_and(e > 0, e < E - 1))
    def _():
        out_ref[...] = out_ref[...] + contrib

    @pl.when(e == E - 1)
    def _():
        acc = out_ref[...] + contrib
        acc = jnp.where(acc == 0.0, np.float32(EPS), acc)
        out_ref[...] = jnp.log(acc)


@jax.jit
def kernel(x, w_gate, fc1_w, fc1_b, fc2_w, fc2_b):
    nt = B // TM
    gates = pl.pallas_call(
        _router_body,
        grid=(nt,),
        in_specs=[
            pl.BlockSpec((TM, D), lambda i: (i, 0)),
            pl.BlockSpec((D, E), lambda i: (0, 0)),
        ],
        out_specs=pl.BlockSpec((TM, E), lambda i: (i, 0)),
        out_shape=jax.ShapeDtypeStruct((B, E), jnp.float32),
    )(x, w_gate)

    xb = x.astype(jnp.bfloat16)
    w1 = fc1_w.astype(jnp.bfloat16)
    w2 = fc2_w.astype(jnp.bfloat16)

    out = pl.pallas_call(
        _moe_body,
        grid=(E, nt),
        in_specs=[
            pl.BlockSpec((TM, D), lambda e, i: (i, 0)),
            pl.BlockSpec((1, D, H), lambda e, i: (e, 0, 0)),
            pl.BlockSpec((1, 1, H), lambda e, i: (e, 0, 0)),
            pl.BlockSpec((1, H, O), lambda e, i: (e, 0, 0)),
            pl.BlockSpec((1, 1, O), lambda e, i: (e, 0, 0)),
            pl.BlockSpec((TM, E), lambda e, i: (i, 0)),
        ],
        out_specs=pl.BlockSpec((TM, O), lambda e, i: (i, 0)),
        out_shape=jax.ShapeDtypeStruct((B, O), jnp.float32),
        compiler_params=pltpu.CompilerParams(
            dimension_semantics=("arbitrary", "arbitrary")
        ),
    )(xb, w1, fc1_b.reshape(E, 1, H), w2, fc2_b.reshape(E, 1, O), gates)
    return out
